# Initial kernel scaffold; baseline (speedup 1.0000x reference)
import jax
import jax.numpy as jnp
from jax.experimental import pallas as pl


def kernel(face_features, codebook):
    raise NotImplementedError("write your pallas kernel here")


# speedup vs baseline: 1.3474x; 1.3474x over previous
import functools

import jax
import jax.numpy as jnp
from jax import lax
from jax.experimental import pallas as pl
from jax.experimental.pallas import tpu as pltpu
from jax.experimental.pallas import tpu_sc as plsc

_NV = 3
_NQ = 2


def _argmin_body(u_ref, cbt_ref, idx_ref):
    _argmin_common(u_ref[...], cbt_ref, idx_ref)


def _argmin_res_body(u_ref, q1_ref, cbt_ref, idx_ref):
    d = u_ref.shape[1]
    _argmin_common(u_ref[...] - q1_ref[:, :d], cbt_ref, idx_ref)


def _argmin_common(r, cbt_ref, idx_ref):
    cbt = cbt_ref[...]
    cn = jnp.sum(cbt * cbt, axis=0, keepdims=True)
    rn = jnp.sum(r * r, axis=1, keepdims=True)
    e = jnp.dot(r, cbt, preferred_element_type=jnp.float32)
    d = rn - 2.0 * e + cn
    mval = jnp.min(d, axis=1, keepdims=True)
    k = d.shape[1]
    iota = lax.broadcasted_iota(jnp.int32, d.shape, 1)
    idx = jnp.min(jnp.where(d == mval, iota, k), axis=1)
    idx_ref[0, 0] = idx


def _argmin_call(u, q1, cbt, tm):
    t, d = u.shape
    k = cbt.shape[1]
    nblk = t // tm
    out_shape = jax.ShapeDtypeStruct((nblk, 1, tm), jnp.int32)
    tok_spec = pl.BlockSpec((tm, d), lambda i: (i, 0))
    cb_spec = pl.BlockSpec((d, k), lambda i: (0, 0))
    idx_spec = pl.BlockSpec((1, 1, tm), lambda i: (i, 0, 0))
    if q1 is None:
        return pl.pallas_call(
            _argmin_body,
            grid=(nblk,),
            in_specs=[tok_spec, cb_spec],
            out_specs=idx_spec,
            out_shape=out_shape,
        )(u, cbt)
    q_spec = pl.BlockSpec((tm, q1.shape[1] if q1 is not None else d),
                          lambda i: (i, 0))
    return pl.pallas_call(
        _argmin_res_body,
        grid=(nblk,),
        in_specs=[tok_spec, q_spec, cb_spec],
        out_specs=idx_spec,
        out_shape=out_shape,
    )(u, q1, cbt)


def _make_sc_gather(t, dpad, chunk):
    info = plsc.get_sparse_core_info()
    nw = info.num_cores * info.num_subcores
    b_per_w = t // nw
    nchunk = b_per_w // chunk
    mesh = plsc.VectorSubcoreMesh(core_axis_name="c", subcore_axis_name="s")

    @functools.partial(
        pl.kernel,
        mesh=mesh,
        out_type=jax.ShapeDtypeStruct((t, dpad), jnp.float32),
        scratch_types=[
            pltpu.VMEM((chunk,), jnp.int32),
            pltpu.VMEM((chunk, dpad), jnp.float32),
            pltpu.SemaphoreType.DMA,
        ],
    )
    def gather_k(cb_hbm, idx_hbm, out_hbm, idx_v, rows_v, sem):
        wid = lax.axis_index("s") * info.num_cores + lax.axis_index("c")
        base = wid * b_per_w
        for c in range(nchunk):
            off = base + c * chunk
            pltpu.sync_copy(idx_hbm.at[pl.ds(off, chunk)], idx_v)
            pltpu.async_copy(cb_hbm.at[idx_v], rows_v, sem).wait()
            pltpu.sync_copy(rows_v, out_hbm.at[pl.ds(off, chunk)])

    return gather_k


def _finish_body(u_ref, q1_ref, q2_ref, quant_ref, l1_ref, l2_ref):
    d = u_ref.shape[1]
    u = u_ref[...]
    q1 = q1_ref[:, :d]
    q2 = q2_ref[:, :d]
    r2 = u - q1
    r3 = r2 - q2
    quant_ref[...] = q1 + q2

    @pl.when(pl.program_id(0) == 0)
    def _():
        l1_ref[...] = jnp.zeros_like(l1_ref)
        l2_ref[...] = jnp.zeros_like(l2_ref)

    l1_ref[...] += jnp.sum(r2 * r2).reshape(1, 1)
    l2_ref[...] += jnp.sum(r3 * r3).reshape(1, 1)


def _finish_call(u, q1, q2, tm):
    t, d = u.shape
    nblk = t // tm
    tok_spec = pl.BlockSpec((tm, d), lambda i: (i, 0))
    q_spec = pl.BlockSpec((tm, q1.shape[1]), lambda i: (i, 0))
    scalar_spec = pl.BlockSpec((1, 1), lambda i: (0, 0))
    return pl.pallas_call(
        _finish_body,
        grid=(nblk,),
        in_specs=[tok_spec, q_spec, q_spec],
        out_specs=[tok_spec, scalar_spec, scalar_spec],
        out_shape=[
            jax.ShapeDtypeStruct((t, d), jnp.float32),
            jax.ShapeDtypeStruct((1, 1), jnp.float32),
            jax.ShapeDtypeStruct((1, 1), jnp.float32),
        ],
    )(u, q1, q2)


def kernel(face_features, codebook):
    b, nf, dim = face_features.shape
    d = dim // _NV
    t = b * nf * _NV
    n = nf * _NV

    u = face_features.reshape(t, d)
    cbt = codebook.T
    dpad = 128
    cb_pad = jnp.pad(codebook, ((0, 0), (0, dpad - d)))

    tm = 256
    gather = _make_sc_gather(t, dpad, chunk=512)

    idx1 = _argmin_call(u, None, cbt, tm).reshape(t)
    q1 = gather(cb_pad, idx1)
    idx2 = _argmin_call(u, q1, cbt, tm).reshape(t)
    q2 = gather(cb_pad, idx2)

    quant, l1, l2 = _finish_call(u, q1, q2, tm=512)

    quantized = quant.reshape(b, nf, dim)
    indices = jnp.stack([idx1.reshape(b, n), idx2.reshape(b, n)], axis=-1)
    commit_loss = jnp.concatenate([l1.reshape(1), l2.reshape(1)]) / float(t * d)
    return quantized, indices, commit_loss


# speedup vs baseline: 1.7308x; 1.2845x over previous
import functools

import jax
import jax.numpy as jnp
from jax import lax
from jax.experimental import pallas as pl
from jax.experimental.pallas import tpu as pltpu
from jax.experimental.pallas import tpu_sc as plsc

_NV = 3
_NQ = 2


def _prep_body(cbt_ref, cbtm2_ref, cn_ref):
    cbt = cbt_ref[...]
    cbtm2_ref[...] = -2.0 * cbt
    cn = jnp.sum(cbt * cbt, axis=0, keepdims=True)
    cn_ref[...] = jnp.broadcast_to(cn, cn_ref.shape)


def _prep_call(cbt):
    d, k = cbt.shape
    return pl.pallas_call(
        _prep_body,
        out_shape=[
            jax.ShapeDtypeStruct((d, k), jnp.float32),
            jax.ShapeDtypeStruct((8, k), jnp.float32),
        ],
    )(cbt)


def _argmin_body(u_ref, cbtm2_ref, cn_ref, idx_ref):
    _argmin_common(u_ref[...], cbtm2_ref, cn_ref, idx_ref)


def _argmin_res_body(u_ref, q1_ref, cbtm2_ref, cn_ref, idx_ref):
    d = u_ref.shape[1]
    _argmin_common(u_ref[...] - q1_ref[:, :d], cbtm2_ref, cn_ref, idx_ref)


def _argmin_common(r, cbtm2_ref, cn_ref, idx_ref):
    tm, d = r.shape
    k = cbtm2_ref.shape[1]

    ck = min(512, k)
    one_bits = jnp.int32(0x3F800000)
    two_bits = jnp.int32(0x40000000)
    best_v = jnp.full((tm, 1), 3.0e38, jnp.float32)
    best_k = jnp.full((tm, 1), 2.0, jnp.float32)
    for c in range(k // ck):
        e2 = jnp.dot(r, cbtm2_ref[:, c * ck:(c + 1) * ck],
                     preferred_element_type=jnp.float32)
        sc = e2 + cn_ref[0:1, c * ck:(c + 1) * ck]
        mv = jnp.min(sc, axis=1, keepdims=True)
        iota = lax.broadcasted_iota(jnp.int32, (tm, ck), 1) + (c * ck)
        cand = jnp.where(sc <= mv, iota | one_bits, two_bits)
        key = jnp.min(lax.bitcast_convert_type(cand, jnp.float32), axis=1,
                      keepdims=True)
        better = mv < best_v
        best_k = jnp.where(better, key, best_k)
        best_v = jnp.where(better, mv, best_v)
    idx = (lax.bitcast_convert_type(best_k[:, 0], jnp.int32)
           & jnp.int32(0x007FFFFF))
    idx_ref[0, 0] = idx


def _argmin_call(u, q1, cbtm2, cn, tm):
    t, d = u.shape
    k = cbtm2.shape[1]
    nblk = t // tm
    out_shape = jax.ShapeDtypeStruct((nblk, 1, tm), jnp.int32)
    tok_spec = pl.BlockSpec((tm, d), lambda i: (i, 0))
    cb_spec = pl.BlockSpec((d, k), lambda i: (0, 0))
    cn_spec = pl.BlockSpec((8, k), lambda i: (0, 0))
    idx_spec = pl.BlockSpec((1, 1, tm), lambda i: (i, 0, 0))
    if q1 is None:
        return pl.pallas_call(
            _argmin_body,
            grid=(nblk,),
            in_specs=[tok_spec, cb_spec, cn_spec],
            out_specs=idx_spec,
            out_shape=out_shape,
        )(u, cbtm2, cn)
    q_spec = pl.BlockSpec((tm, q1.shape[1]), lambda i: (i, 0))
    return pl.pallas_call(
        _argmin_res_body,
        grid=(nblk,),
        in_specs=[tok_spec, q_spec, cb_spec, cn_spec],
        out_specs=idx_spec,
        out_shape=out_shape,
    )(u, q1, cbtm2, cn)


def _make_sc_gather(t, dpad, chunk):
    info = plsc.get_sparse_core_info()
    nw = info.num_cores * info.num_subcores
    b_per_w = t // nw
    nchunk = b_per_w // chunk
    mesh = plsc.VectorSubcoreMesh(core_axis_name="c", subcore_axis_name="s")

    @functools.partial(
        pl.kernel,
        mesh=mesh,
        out_type=jax.ShapeDtypeStruct((t, dpad), jnp.float32),
        scratch_types=[
            pltpu.VMEM((chunk,), jnp.int32),
            pltpu.VMEM((chunk, dpad), jnp.float32),
            pltpu.SemaphoreType.DMA,
        ],
    )
    def gather_k(cb_hbm, idx_hbm, out_hbm, idx_v, rows_v, sem):
        wid = lax.axis_index("s") * info.num_cores + lax.axis_index("c")
        base = wid * b_per_w
        for c in range(nchunk):
            off = base + c * chunk
            pltpu.sync_copy(idx_hbm.at[pl.ds(off, chunk)], idx_v)
            pltpu.async_copy(cb_hbm.at[idx_v], rows_v, sem).wait()
            pltpu.sync_copy(rows_v, out_hbm.at[pl.ds(off, chunk)])

    return gather_k


def _finish_body(u_ref, q1_ref, q2_ref, quant_ref, l1_ref, l2_ref):
    d = u_ref.shape[1]
    u = u_ref[...]
    q1 = q1_ref[:, :d]
    q2 = q2_ref[:, :d]
    r2 = u - q1
    r3 = r2 - q2
    quant_ref[...] = q1 + q2

    @pl.when(pl.program_id(0) == 0)
    def _():
        l1_ref[...] = jnp.zeros_like(l1_ref)
        l2_ref[...] = jnp.zeros_like(l2_ref)

    l1_ref[...] += jnp.sum(r2 * r2).reshape(1, 1)
    l2_ref[...] += jnp.sum(r3 * r3).reshape(1, 1)


def _finish_call(u, q1, q2, tm):
    t, d = u.shape
    nblk = t // tm
    tok_spec = pl.BlockSpec((tm, d), lambda i: (i, 0))
    q_spec = pl.BlockSpec((tm, q1.shape[1]), lambda i: (i, 0))
    scalar_spec = pl.BlockSpec((1, 1), lambda i: (0, 0))
    return pl.pallas_call(
        _finish_body,
        grid=(nblk,),
        in_specs=[tok_spec, q_spec, q_spec],
        out_specs=[tok_spec, scalar_spec, scalar_spec],
        out_shape=[
            jax.ShapeDtypeStruct((t, d), jnp.float32),
            jax.ShapeDtypeStruct((1, 1), jnp.float32),
            jax.ShapeDtypeStruct((1, 1), jnp.float32),
        ],
    )(u, q1, q2)


def kernel(face_features, codebook):
    b, nf, dim = face_features.shape
    d = dim // _NV
    t = b * nf * _NV
    n = nf * _NV

    u = face_features.reshape(t, d)
    cbt = codebook.T
    dpad = 128
    cb_pad = jnp.pad(codebook, ((0, 0), (0, dpad - d)))

    tm = 512
    gather = _make_sc_gather(t, dpad, chunk=512)

    cbtm2, cn = _prep_call(cbt)
    idx1 = _argmin_call(u, None, cbtm2, cn, tm).reshape(t)
    q1 = gather(cb_pad, idx1)
    idx2 = _argmin_call(u, q1, cbtm2, cn, tm).reshape(t)
    q2 = gather(cb_pad, idx2)

    quant, l1, l2 = _finish_call(u, q1, q2, tm=512)

    quantized = quant.reshape(b, nf, dim)
    indices = jnp.stack([idx1.reshape(b, n), idx2.reshape(b, n)], axis=-1)
    commit_loss = jnp.concatenate([l1.reshape(1), l2.reshape(1)]) / float(t * d)
    return quantized, indices, commit_loss
